# baseline (speedup 1.0000x reference)
import jax
import jax.numpy as jnp
from jax.experimental import pallas as pl
from jax.experimental.pallas import tpu as pltpu
from jax.sharding import Mesh, PartitionSpec as P

_S = 8
_L = 128
_LOGP = 18


def _make_loss_kernel(g_len, t):
    lg = 11 - (t.bit_length() - 1)
    assert g_len == 1 << lg

    def body(logits_ref, labels_ref, out_ref):
        x = logits_ref[...]
        lab = labels_ref[...]
        labf = lab.astype(jnp.float32)
        e = 1.0 - x * (2.0 * labf - 1.0)

        bits = pltpu.bitcast(e, jnp.int32)
        key = jnp.where(bits >= 0, bits, bits ^ jnp.int32(0x7FFFFFFF))
        key = (key & jnp.int32(-2)) | lab

        l_i = jax.lax.broadcasted_iota(jnp.int32, (1, 1, _L), 2)
        s_i = jax.lax.broadcasted_iota(jnp.int32, (1, _S, 1), 1)
        g_i0 = jax.lax.broadcasted_iota(jnp.int32, (g_len, 1, 1), 0)

        def lane_bit(b):
            return ((l_i >> (b - 11)) & 1) == 1

        def flip_mask(ke):
            if ke < lg:
                return ((g_i0 >> ke) & 1) - jnp.int32(1)
            if ke < 11:
                return ((s_i >> (ke - lg)) & 1) - jnp.int32(1)
            if ke <= 17:
                return ((l_i >> (ke - 11)) & 1) - jnp.int32(1)
            return jnp.int32(-1)

        n = 16
        w = key.reshape(g_len // n, n, _S, _L)
        parts = [w[:, i] for i in range(n)]
        for ke in range(1, 4):
            for jl in range(ke - 1, -1, -1):
                d = 1 << jl
                for i in range(n):
                    if not (i & d):
                        asc = ((i >> ke) & 1) == 1
                        a, b = parts[i], parts[i + d]
                        lo = jnp.minimum(a, b)
                        hi = jnp.maximum(a, b)
                        parts[i], parts[i + d] = (lo, hi) if asc else (hi, lo)
        key = jnp.concatenate(
            [p[:, None] for p in parts], axis=1).reshape(g_len, _S, _L)

        def low_merge(key, kbits):
            n = 1 << kbits
            w = key.reshape(g_len // n, n, _S, _L)
            parts = [w[:, i] for i in range(n)]
            for jl in range(kbits - 1, -1, -1):
                d = 1 << jl
                for i in range(n):
                    if not (i & d):
                        a, b = parts[i], parts[i + d]
                        parts[i] = jnp.minimum(a, b)
                        parts[i + d] = jnp.maximum(a, b)
            return jnp.concatenate(
                [p[:, None] for p in parts], axis=1).reshape(g_len, _S, _L)

        _LOW = 5
        key = key ^ flip_mask(4)
        for ke in range(4, _LOGP + 1):
            for jl in range(ke - 1, 10, -1):
                d = 1 << (jl - 11)
                bit_u = lane_bit(jl)
                y = pltpu.roll(key, _L - d, 2)
                lo = jnp.minimum(key, y)
                hi = jnp.maximum(key, y)
                key = jnp.where(bit_u, pltpu.roll(hi, d, 2), lo)
            for jl in range(min(ke - 1, 10), lg - 1, -1):
                d = 1 << (jl - lg)
                bit_u = ((s_i >> (jl - lg)) & 1) == 1
                y = pltpu.roll(key, _S - d, 1)
                lo = jnp.minimum(key, y)
                hi = jnp.maximum(key, y)
                key = jnp.where(bit_u, pltpu.roll(hi, d, 1), lo)
            hi_top = min(ke - 1, lg - 1)
            if hi_top >= _LOW:
                chunk = g_len // 64
                cb = chunk.bit_length() - 1
                w = key.reshape(64, chunk, _S, _L)
                hparts = [w[i] for i in range(64)]
                for jl in range(hi_top, _LOW - 1, -1):
                    pd = 1 << (jl - cb)
                    for i in range(64):
                        if not (i & pd):
                            a, b = hparts[i], hparts[i + pd]
                            hparts[i] = jnp.minimum(a, b)
                            hparts[i + pd] = jnp.maximum(a, b)
                key = jnp.concatenate(
                    [p[None] for p in hparts], axis=0).reshape(g_len, _S, _L)
            key = low_merge(key, min(ke, _LOW))
            if ke < _LOGP:
                key = key ^ (flip_mask(ke) ^ flip_mask(ke + 1))
            else:
                key = key ^ flip_mask(ke)

        gt = (key & 1).astype(jnp.float32)
        ebits = jnp.where(key >= 0, key, key ^ jnp.int32(0x7FFFFFFF))
        relu_e = jnp.maximum(pltpu.bitcast(ebits, jnp.float32), 0.0)

        wg = gt.reshape(g_len // 16, 16, _S, _L)
        qparts = [wg[:, i] for i in range(16)]
        for i in range(1, 16):
            qparts[i] = qparts[i] + qparts[i - 1]
        t1 = qparts[15]
        e1 = jnp.concatenate(
            [jnp.zeros((1, _S, _L), jnp.float32), t1[:-1]], axis=0)
        sh = 1
        while sh < g_len // 16:
            e1 = e1 + jnp.concatenate(
                [jnp.zeros((sh, _S, _L), jnp.float32), e1[:-sh]], axis=0)
            sh *= 2
        c = jnp.concatenate(
            [(q + e1)[:, None] for q in qparts], axis=1
        ).reshape(g_len, _S, _L)

        tt = c[g_len - 1]
        s_col = jax.lax.broadcasted_iota(jnp.int32, (_S, 1), 0)
        ps = jnp.zeros((_S, _L), jnp.float32)
        tg = tt
        for j in range(t.bit_length() - 1):
            d = 1 << j
            up = ((s_col >> j) & 1) == 1
            dn_roll = pltpu.roll(tg, d, 0)
            up_roll = pltpu.roll(tg, _S - d, 0)
            pair = jnp.where(up, dn_roll, up_roll)
            ps = ps + jnp.where(up, pltpu.roll(ps, d, 0) + dn_roll, 0.0)
            tg = tg + pair
        ii = jax.lax.broadcasted_iota(jnp.int32, (_L, _L), 0)
        jj = jax.lax.broadcasted_iota(jnp.int32, (_L, _L), 1)
        lts = (ii < jj).astype(jnp.float32)
        lp = jax.lax.dot(tg, lts, precision=jax.lax.Precision.HIGHEST)
        cum_gt = c + ps[None] + lp[None]
        g_total = jnp.sum(tg, axis=1)[None, :, None]

        sub_i = s_i & (t - 1)
        pos = (g_i0 + sub_i * g_len + l_i * (g_len * t) + 1).astype(
            jnp.float32)
        inter = g_total - cum_gt
        union = g_total + pos - cum_gt
        jacc = 1.0 - inter / union

        a = jnp.concatenate([jacc[g_len - 1:], jacc[:-1]], axis=0)
        if t > 1:
            b2 = pltpu.roll(a, 1, 1)
            c2 = pltpu.roll(pltpu.roll(a, _S - (t - 1), 1), 1, 2)
            prev = jnp.where(sub_i == 0, c2, b2)
            prev = jnp.where(g_i0 == 0, prev, a)
        else:
            b2 = pltpu.roll(a, 1, 2)
            prev = jnp.where(g_i0 == 0, b2, a)
        prev = jnp.where((g_i0 == 0) & (sub_i == 0) & (l_i == 0), 0.0, prev)
        contrib = relu_e * (jacc - prev)
        part = jnp.sum(contrib, axis=0)
        for j in range(t.bit_length() - 1):
            d = 1 << j
            up = ((s_col >> j) & 1) == 1
            part = part + jnp.where(up, pltpu.roll(part, d, 0),
                                    pltpu.roll(part, _S - d, 0))
        ones_l = jnp.ones((_L, _L), jnp.float32)
        img_tot = jax.lax.dot(part, ones_l,
                              precision=jax.lax.Precision.HIGHEST)
        out_ref[...] = img_tot

    return body


def _shard_loss(logits, labels, t):
    m = 8 // t
    g_len = 1 << (11 - (t.bit_length() - 1))
    lf = jnp.transpose(
        logits.reshape(m, t, g_len, _L), (2, 0, 1, 3)).reshape(g_len, _S, _L)
    lb = jnp.transpose(
        labels.reshape(m, t, g_len, _L), (2, 0, 1, 3)).reshape(g_len, _S, _L)
    out = pl.pallas_call(
        _make_loss_kernel(g_len, t),
        in_specs=[
            pl.BlockSpec((g_len, _S, _L), lambda: (0, 0, 0)),
            pl.BlockSpec((g_len, _S, _L), lambda: (0, 0, 0)),
        ],
        out_specs=pl.BlockSpec((_S, _L), lambda: (0, 0)),
        out_shape=jax.ShapeDtypeStruct((_S, _L), jnp.float32),
    )(lf, lb)
    return out[::t, 0]


def kernel(logits, labels):
    devs = jax.devices()
    n = 2 if len(devs) >= 2 else 1
    if n == 1:
        return jnp.mean(_shard_loss(logits, labels, 1))
    mesh = Mesh(devs[:n], ("b",))
    f = jax.shard_map(
        lambda lx, lb: _shard_loss(lx, lb, n),
        mesh=mesh, in_specs=(P("b"), P("b")), out_specs=P("b"),
        check_vma=False)
    return jnp.mean(f(logits, labels))


# speedup vs baseline: 2.1880x; 2.1880x over previous
import jax
import jax.numpy as jnp
from jax.experimental import pallas as pl
from jax.experimental.pallas import tpu as pltpu

_G = 2048
_S = 8
_L = 128
_LOGP = 18


def _loss_kernel(logits_ref, labels_ref, out_ref):
    x = logits_ref[...]
    lab = labels_ref[...]
    labf = lab.astype(jnp.float32)
    e = 1.0 - x * (2.0 * labf - 1.0)

    bits = pltpu.bitcast(e, jnp.int32)
    key = jnp.where(bits >= 0, bits, bits ^ jnp.int32(0x7FFFFFFF))
    key = (key & jnp.int32(-2)) | lab

    l_i = jax.lax.broadcasted_iota(jnp.int32, (1, 1, _L), 2)
    g_i0 = jax.lax.broadcasted_iota(jnp.int32, (_G, 1, 1), 0)

    def lane_bit(b):
        return ((l_i >> (b - 11)) & 1) == 1

    def flip_mask(ke):
        if ke <= 10:
            return ((g_i0 >> ke) & 1) - jnp.int32(1)
        if ke <= 17:
            return ((l_i >> (ke - 11)) & 1) - jnp.int32(1)
        return jnp.int32(-1)

    def low_merge(key, kbits):
        n = 1 << kbits
        w = key.reshape(_G // n, n, _S, _L)
        parts = [w[:, i] for i in range(n)]
        for jl in range(kbits - 1, -1, -1):
            d = 1 << jl
            for i in range(n):
                if not (i & d):
                    a, b = parts[i], parts[i + d]
                    parts[i] = jnp.minimum(a, b)
                    parts[i + d] = jnp.maximum(a, b)
        return jnp.concatenate(
            [p[:, None] for p in parts], axis=1).reshape(_G, _S, _L)

    n = 16
    w = key.reshape(_G // n, n, _S, _L)
    parts = [w[:, i] for i in range(n)]
    for ke in range(1, 4):
        for jl in range(ke - 1, -1, -1):
            d = 1 << jl
            for i in range(n):
                if not (i & d):
                    asc = ((i >> ke) & 1) == 1
                    a, b = parts[i], parts[i + d]
                    lo = jnp.minimum(a, b)
                    hi = jnp.maximum(a, b)
                    parts[i], parts[i + d] = (lo, hi) if asc else (hi, lo)
    key = jnp.concatenate(
        [p[:, None] for p in parts], axis=1).reshape(_G, _S, _L)

    _LOW = 5
    key = key ^ flip_mask(4)
    for ke in range(4, _LOGP + 1):
        for jl in range(ke - 1, 10, -1):
            d = 1 << (jl - 11)
            bit_u = lane_bit(jl)
            y = pltpu.roll(key, _L - d, 2)
            lo = jnp.minimum(key, y)
            hi = jnp.maximum(key, y)
            key = jnp.where(bit_u, pltpu.roll(hi, d, 2), lo)
        hi_top = min(ke - 1, 10)
        if hi_top >= _LOW:
            w = key.reshape(64, _G // 64, _S, _L)
            hparts = [w[i] for i in range(64)]
            for jl in range(hi_top, _LOW - 1, -1):
                pd = 1 << (jl - _LOW)
                for i in range(64):
                    if not (i & pd):
                        a, b = hparts[i], hparts[i + pd]
                        hparts[i] = jnp.minimum(a, b)
                        hparts[i + pd] = jnp.maximum(a, b)
            key = jnp.concatenate(
                [p[None] for p in hparts], axis=0).reshape(_G, _S, _L)
        key = low_merge(key, min(ke, _LOW))
        if ke < _LOGP:
            key = key ^ (flip_mask(ke) ^ flip_mask(ke + 1))
        else:
            key = key ^ flip_mask(ke)

    gt = (key & 1).astype(jnp.float32)
    ebits = jnp.where(key >= 0, key, key ^ jnp.int32(0x7FFFFFFF))
    relu_e = jnp.maximum(pltpu.bitcast(ebits, jnp.float32), 0.0)

    wg = gt.reshape(_G // 16, 16, _S, _L)
    qparts = [wg[:, i] for i in range(16)]
    for i in range(1, 16):
        qparts[i] = qparts[i] + qparts[i - 1]
    t1 = qparts[15]
    e1 = jnp.concatenate(
        [jnp.zeros((1, _S, _L), jnp.float32), t1[:-1]], axis=0)
    sh = 1
    while sh < _G // 16:
        e1 = e1 + jnp.concatenate(
            [jnp.zeros((sh, _S, _L), jnp.float32), e1[:-sh]], axis=0)
        sh *= 2
    c = jnp.concatenate(
        [(q + e1)[:, None] for q in qparts], axis=1).reshape(_G, _S, _L)
    t = c[_G - 1]
    ii = jax.lax.broadcasted_iota(jnp.int32, (_L, _L), 0)
    jj = jax.lax.broadcasted_iota(jnp.int32, (_L, _L), 1)
    lts = (ii < jj).astype(jnp.float32)
    p1 = jax.lax.dot(t, lts, precision=jax.lax.Precision.HIGHEST)
    cum_gt = c + p1[None]
    g_total = jnp.sum(t, axis=1)[None, :, None]

    g_i = jax.lax.broadcasted_iota(jnp.int32, (_G, 1, 1), 0)
    pos = (g_i + l_i * _G + 1).astype(jnp.float32)
    inter = g_total - cum_gt
    union = g_total + pos - cum_gt
    jacc = 1.0 - inter / union

    a = jnp.concatenate([jacc[_G - 1:], jacc[:-1]], axis=0)
    b2 = pltpu.roll(a, 1, 2)
    prev = jnp.where(g_i == 0, b2, a)
    prev = jnp.where((g_i == 0) & (l_i == 0), 0.0, prev)
    contrib = relu_e * (jacc - prev)
    part = jnp.sum(contrib, axis=0)
    ones_l = jnp.ones((_L, _L), jnp.float32)
    img_tot = jax.lax.dot(part, ones_l,
                          precision=jax.lax.Precision.HIGHEST)
    out_ref[...] = img_tot


def kernel(logits, labels):
    lf = jnp.transpose(logits.reshape(_S, _G, _L), (1, 0, 2))
    lb = jnp.transpose(labels.reshape(_S, _G, _L), (1, 0, 2))
    losses = pl.pallas_call(
        _loss_kernel,
        in_specs=[
            pl.BlockSpec((_G, _S, _L), lambda: (0, 0, 0)),
            pl.BlockSpec((_G, _S, _L), lambda: (0, 0, 0)),
        ],
        out_specs=pl.BlockSpec((_S, _L), lambda: (0, 0)),
        out_shape=jax.ShapeDtypeStruct((_S, _L), jnp.float32),
    )(lf, lb)
    return jnp.mean(losses[:, 0])
